# baseline (speedup 1.0000x reference)
import jax
import jax.numpy as jnp
from jax import lax
from jax.experimental import pallas as pl
from jax.experimental.pallas import tpu as pltpu
from jax.experimental.pallas import tpu_sc as plsc

_N = 32768
_D = 256
_B = 16
_NC = 2
_NS = 16
_NW = _NC * _NS
_RPW = _N // _NW
_CH = 64
_NBUF = 4
_NCHUNK = _RPW // _CH


def _copy_body(nf_hbm, fv_hbm, b0, b1, b2, b3, rsem, wsem):
    bufs = (b0, b1, b2, b3)
    cid = lax.axis_index("c")
    sid = lax.axis_index("s")
    wid = sid * _NC + cid
    base = wid * _RPW

    def rd(i):
        b = i % _NBUF
        return pltpu.make_async_copy(
            nf_hbm.at[pl.ds(base + i * _CH, _CH), :], bufs[b], rsem.at[b])

    def wr(i):
        b = i % _NBUF
        return pltpu.make_async_copy(
            bufs[b], fv_hbm.at[pl.ds(base + i * _CH, _CH), :], wsem.at[b])

    for i in range(_NBUF):
        rd(i).start()
    last_waited_w = -1
    for i in range(_NCHUNK):
        rd(i).wait()
        wr(i).start()
        j = i - 2
        if j >= 0 and j + _NBUF < _NCHUNK:
            wr(j).wait()
            last_waited_w = j
            rd(j + _NBUF).start()
    for i in range(last_waited_w + 1, _NCHUNK):
        wr(i).wait()


def _counts_body(gi_ref, cnt_ref, mask_ref):
    gi = gi_ref[...]
    ranks = []
    for t in range(1, _B + 1):
        ranks.append(jnp.sum((gi < t).astype(jnp.int32)))
    cnts = []
    prev = jnp.int32(0)
    for t in range(_B):
        cnts.append(ranks[t] - prev)
        prev = ranks[t]
    c = jnp.stack(cnts)
    cnt_ref[...] = c
    mask_ref[...] = c > 0


@jax.jit
def _run(node_features, graph_indicator):
    mesh = plsc.VectorSubcoreMesh(core_axis_name="c", subcore_axis_name="s")
    flat_values = pl.kernel(
        _copy_body,
        out_type=jax.ShapeDtypeStruct((_N, _D), jnp.float32),
        mesh=mesh,
        scratch_types=[
            pltpu.VMEM((_CH, _D), jnp.float32),
            pltpu.VMEM((_CH, _D), jnp.float32),
            pltpu.VMEM((_CH, _D), jnp.float32),
            pltpu.VMEM((_CH, _D), jnp.float32),
            pltpu.SemaphoreType.DMA((_NBUF,)),
            pltpu.SemaphoreType.DMA((_NBUF,)),
        ],
        compiler_params=pltpu.CompilerParams(needs_layout_passes=False),
    )(node_features)

    counts, mask = pl.pallas_call(
        _counts_body,
        in_specs=[pl.BlockSpec((_N // 128, 128), lambda: (0, 0))],
        out_specs=[
            pl.BlockSpec((_B,), lambda: (0,)),
            pl.BlockSpec((_B,), lambda: (0,)),
        ],
        out_shape=[
            jax.ShapeDtypeStruct((_B,), jnp.int32),
            jax.ShapeDtypeStruct((_B,), jnp.bool_),
        ],
    )(graph_indicator.reshape(_N // 128, 128))
    return flat_values, counts, mask


def kernel(node_features, graph_indicator):
    return _run(node_features, graph_indicator)


# speedup vs baseline: 1.0219x; 1.0219x over previous
import jax
import jax.numpy as jnp
from jax import lax
from jax.experimental import pallas as pl
from jax.experimental.pallas import tpu as pltpu
from jax.experimental.pallas import tpu_sc as plsc

_N = 32768
_D = 256
_B = 16
_NC = 2
_NS = 16
_NW = _NC * _NS
_RPW = _N // _NW
_CH = 64
_NBUF = 6
_NCHUNK = _RPW // _CH
_LEAD = 3


def _copy_body(nf_hbm, fv_hbm, b0, b1, b2, b3, b4, b5, rsem, wsem):
    bufs = (b0, b1, b2, b3, b4, b5)
    cid = lax.axis_index("c")
    sid = lax.axis_index("s")
    wid = sid * _NC + cid
    base = wid * _RPW

    def rd(i):
        b = i % _NBUF
        return pltpu.make_async_copy(
            nf_hbm.at[pl.ds(base + i * _CH, _CH), :], bufs[b], rsem.at[b])

    def wr(i):
        b = i % _NBUF
        return pltpu.make_async_copy(
            bufs[b], fv_hbm.at[pl.ds(base + i * _CH, _CH), :], wsem.at[b])

    waited_w = -1
    for i in range(_LEAD):
        rd(i).start()
    for i in range(_NCHUNK):
        rd(i).wait()
        wr(i).start()
        if i + _LEAD < _NCHUNK:
            j = i - (_NBUF - _LEAD)
            if j >= 0:
                wr(j).wait()
                waited_w = j
            rd(i + _LEAD).start()
    for i in range(waited_w + 1, _NCHUNK):
        wr(i).wait()


def _counts_body(gi_ref, cnt_ref, mask_ref):
    gi = gi_ref[...]
    ranks = []
    for t in range(1, _B + 1):
        ranks.append(jnp.sum((gi < t).astype(jnp.int32)))
    cnts = []
    prev = jnp.int32(0)
    for t in range(_B):
        cnts.append(ranks[t] - prev)
        prev = ranks[t]
    c = jnp.stack(cnts)
    cnt_ref[...] = c
    mask_ref[...] = c > 0


@jax.jit
def _run(node_features, graph_indicator):
    mesh = plsc.VectorSubcoreMesh(core_axis_name="c", subcore_axis_name="s")
    flat_values = pl.kernel(
        _copy_body,
        out_type=jax.ShapeDtypeStruct((_N, _D), jnp.float32),
        mesh=mesh,
        scratch_types=(
            [pltpu.VMEM((_CH, _D), jnp.float32)] * _NBUF
            + [
                pltpu.SemaphoreType.DMA((_NBUF,)),
                pltpu.SemaphoreType.DMA((_NBUF,)),
            ]
        ),
        compiler_params=pltpu.CompilerParams(needs_layout_passes=False),
    )(node_features)

    counts, mask = pl.pallas_call(
        _counts_body,
        in_specs=[pl.BlockSpec((_N // 128, 128), lambda: (0, 0))],
        out_specs=[
            pl.BlockSpec((_B,), lambda: (0,)),
            pl.BlockSpec((_B,), lambda: (0,)),
        ],
        out_shape=[
            jax.ShapeDtypeStruct((_B,), jnp.int32),
            jax.ShapeDtypeStruct((_B,), jnp.bool_),
        ],
    )(graph_indicator.reshape(_N // 128, 128))
    return flat_values, counts, mask


def kernel(node_features, graph_indicator):
    return _run(node_features, graph_indicator)


# speedup vs baseline: 1.1657x; 1.1407x over previous
import jax
import jax.numpy as jnp
from jax import lax
from jax.experimental import pallas as pl
from jax.experimental.pallas import tpu as pltpu
from jax.experimental.pallas import tpu_sc as plsc

_N = 32768
_D = 256
_B = 16
_NC = 2
_COPY_BLOCK = 8192


def _count_body(gi_hbm, counts_hbm, ids_v, cnt_v):
    cid = lax.axis_index("c")
    sid = lax.axis_index("s")
    wid = sid * _NC + cid

    @pl.when(wid == 0)
    def _():
        pltpu.sync_copy(gi_hbm, ids_v)
        lane = lax.iota(jnp.int32, 16)
        t = lane + 1
        lb = jnp.zeros((16,), jnp.int32)
        for k in range(15, -1, -1):
            s = 1 << k
            cand = lb + s
            idx = jnp.minimum(cand, _N) - 1
            vals = plsc.load_gather(ids_v, [idx])
            ok = (cand <= _N) & (vals < t)
            lb = jnp.where(ok, cand, lb)
        cnt_v[...] = lb
        prev = plsc.load_gather(cnt_v, [jnp.maximum(lane - 1, 0)])
        prev = jnp.where(lane == 0, 0, prev)
        cnt_v[...] = lb - prev
        pltpu.sync_copy(cnt_v, counts_hbm)


def _copy_body(nf_ref, out_ref):
    out_ref[...] = nf_ref[...]


@jax.jit
def _run(node_features, graph_indicator):
    mesh = plsc.VectorSubcoreMesh(core_axis_name="c", subcore_axis_name="s")
    counts = pl.kernel(
        _count_body,
        out_type=jax.ShapeDtypeStruct((_B,), jnp.int32),
        mesh=mesh,
        scratch_types=[
            pltpu.VMEM((_N,), jnp.int32),
            pltpu.VMEM((_B,), jnp.int32),
        ],
        compiler_params=pltpu.CompilerParams(needs_layout_passes=False),
    )(graph_indicator)

    flat_values = pl.pallas_call(
        _copy_body,
        grid=(_N // _COPY_BLOCK,),
        in_specs=[pl.BlockSpec((_COPY_BLOCK, _D), lambda i: (i, 0))],
        out_specs=pl.BlockSpec((_COPY_BLOCK, _D), lambda i: (i, 0)),
        out_shape=jax.ShapeDtypeStruct((_N, _D), jnp.float32),
    )(node_features)
    return flat_values, counts


def kernel(node_features, graph_indicator):
    flat_values, counts = _run(node_features, graph_indicator)
    return flat_values, counts, counts > 0
